# baseline (speedup 1.0000x reference)
import jax
import jax.numpy as jnp
from jax import lax
from jax.experimental import pallas as pl
from jax.experimental.pallas import tpu as pltpu
from jax.experimental.pallas import tpu_sc as plsc


def _bcast_take(vec, idx):
    return lax.gather(
        vec, idx[:, None],
        lax.GatherDimensionNumbers(offset_dims=(), collapsed_slice_dims=(0,),
                                   start_index_map=(0,)),
        (1,), mode=lax.GatherScatterMode.PROMISE_IN_BOUNDS)


def _build_sc_kernel(nc, wpb, n_pts, per_w, n_it_last, rows_per_batch,
                     tot_cols):
    CH = 16
    HC = tot_cols // 2
    n_it_full = per_w // CH
    mesh = plsc.VectorSubcoreMesh(core_axis_name="c", subcore_axis_name="s")

    def body(xs_hbm, ys_hbm, zs_hbm, coef_hbm, feat0_hbm, feat1_hbm, out_hbm,
             xs_v, ys_v, zs_v, coef_v, idx_v, rows0_v, rows1_v, out_v,
             sem0, sem1):
        wid = lax.axis_index("s") * nc + lax.axis_index("c")
        b = wid // wpb
        r = wid % wpb
        base = b * n_pts + r * per_w
        n_it = jnp.where(r == wpb - 1, n_it_last, n_it_full)
        pltpu.sync_copy(xs_hbm.at[pl.ds(base, per_w)], xs_v)
        pltpu.sync_copy(ys_hbm.at[pl.ds(base, per_w)], ys_v)
        pltpu.sync_copy(zs_hbm.at[pl.ds(base, per_w)], zs_v)
        pltpu.sync_copy(coef_hbm, coef_v)
        cvec = coef_v[pl.ds(b * 16, 16)]

        def bc(i):
            return _bcast_take(cvec, jnp.full((16,), i, jnp.int32))

        av = bc(0)
        bv = bc(1)
        cv = bc(2)
        dv = bc(3)
        ev = bc(4)
        rowoff = b * rows_per_batch

        def stage_a(o, par):
            x = xs_v[pl.ds(o, 16)]
            y = ys_v[pl.ds(o, 16)]
            z = zs_v[pl.ds(o, 16)] - 0.8
            wn = (av * x + bv * y) / z + cv
            hn = (dv * y) / z + ev
            wn = jnp.minimum(jnp.maximum(wn, -1.0), 1.0)
            hn = jnp.minimum(jnp.maximum(hn, -1.0), 1.0)
            ixf = ((wn + 1.0) * 56.0 - 1.0) * 0.5
            iyf = ((hn + 1.0) * 56.0 - 1.0) * 0.5

            def fl(v):
                t = v.astype(jnp.int32)
                tf = t.astype(jnp.float32)
                return t - (tf > v).astype(jnp.int32)

            ix0 = fl(ixf)
            iy0 = fl(iyf)
            wx1 = ixf - ix0.astype(jnp.float32)
            wx0 = 1.0 - wx1
            wy1 = iyf - iy0.astype(jnp.float32)
            wy0 = 1.0 - wy1
            ix1 = ix0 + 1
            iy1 = iy0 + 1
            zero = jnp.zeros((16,), jnp.int32)
            last = jnp.full((16,), 55, jnp.int32)
            vx0 = ((ix0 >= 0) & (ix0 <= 55)).astype(jnp.float32)
            vx1 = ((ix1 >= 0) & (ix1 <= 55)).astype(jnp.float32)
            vy0 = ((iy0 >= 0) & (iy0 <= 55)).astype(jnp.float32)
            vy1 = ((iy1 >= 0) & (iy1 <= 55)).astype(jnp.float32)
            cx0 = jnp.minimum(jnp.maximum(ix0, zero), last)
            cx1 = jnp.minimum(jnp.maximum(ix1, zero), last)
            cy0 = jnp.minimum(jnp.maximum(iy0, zero), last)
            cy1 = jnp.minimum(jnp.maximum(iy1, zero), last)
            idx_v[par, pl.ds(0, 16)] = rowoff + cy0 * 56 + cx0
            idx_v[par, pl.ds(16, 16)] = rowoff + cy0 * 56 + cx1
            idx_v[par, pl.ds(32, 16)] = rowoff + cy1 * 56 + cx0
            idx_v[par, pl.ds(48, 16)] = rowoff + cy1 * 56 + cx1
            return (wy0 * wx0 * vy0 * vx0, wy0 * wx1 * vy0 * vx1,
                    wy1 * wx0 * vy1 * vx0, wy1 * wx1 * vy1 * vx1)

        def compute_half(ws, rows_ref, hoff):
            w00, w01, w10, w11 = ws

            def p_body(p, c2):
                lane = jnp.full((16,), 0, jnp.int32) + p
                wb0 = _bcast_take(w00, lane)
                wb1 = _bcast_take(w01, lane)
                wb2 = _bcast_take(w10, lane)
                wb3 = _bcast_take(w11, lane)
                obase = p * tot_cols + hoff

                @plsc.parallel_loop(0, HC // 16, unroll=4)
                def s_body(s):
                    off = s * 16
                    acc = (wb0 * rows_ref[p, pl.ds(off, 16)]
                           + wb1 * rows_ref[p + 16, pl.ds(off, 16)]
                           + wb2 * rows_ref[p + 32, pl.ds(off, 16)]
                           + wb3 * rows_ref[p + 48, pl.ds(off, 16)])
                    out_v[pl.ds(obase + off, 16)] = acc

                return c2

            lax.fori_loop(0, CH, p_body, 0)

        ws0 = stage_a(0, 0)
        pltpu.async_copy(feat0_hbm.at[idx_v.at[0]], rows0_v, sem0)

        def iter_body(it, ws):
            par = lax.rem(it, 2)
            o = it * CH
            h1 = pltpu.async_copy(feat1_hbm.at[idx_v.at[par]], rows1_v, sem1)
            o_next = jnp.minimum(it + 1, n_it - 1) * CH
            ws_next = stage_a(o_next, 1 - par)
            pltpu.make_async_copy(
                feat0_hbm.at[idx_v.at[par]], rows0_v, sem0).wait()
            compute_half(ws, rows0_v, 0)
            pltpu.async_copy(feat0_hbm.at[idx_v.at[1 - par]], rows0_v, sem0)
            h1.wait()
            compute_half(ws, rows1_v, HC)
            pltpu.sync_copy(
                out_v, out_hbm.at[pl.ds((base + o) * tot_cols, CH * tot_cols)])
            return ws_next

        lax.fori_loop(0, n_it, iter_body, ws0)
        pltpu.make_async_copy(
            feat0_hbm.at[idx_v.at[lax.rem(n_it, 2)]], rows0_v, sem0).wait()

    return mesh, body


def kernel(resolution, img_features, inputs, camK):
    B, N, _ = inputs.shape
    L, _, C, H, W = img_features.shape
    CT = L * C

    info = plsc.get_sparse_core_info()
    NC, NS = info.num_cores, info.num_subcores
    NW = NC * NS
    wpb = NW // B
    per_w = -(-N // (wpb * 16)) * 16
    per_w_last = N - (wpb - 1) * per_w
    assert per_w_last > 0 and per_w_last % 16 == 0

    scale = 256.0 / 1920.0
    k = camK * scale
    hr = (resolution - 1.0) / 2.0
    hr0, hr1 = hr[0], hr[1]
    a = -k[:, 0, 0] / hr0
    bb = -k[:, 0, 1] / hr0
    c = (k[:, 0, 2] - hr0) / hr0
    d = k[:, 1, 1] / hr1
    e = (k[:, 1, 2] - hr1) / hr1
    z3 = jnp.zeros_like(a)
    coef = jnp.stack([a, bb, c, d, e] + [z3] * 11, axis=1).reshape(-1)

    feat = jnp.transpose(img_features, (1, 3, 4, 0, 2)).reshape(B * H * W, CT)
    feat0 = feat[:, :CT // 2]
    feat1 = feat[:, CT // 2:]

    xs = jnp.pad(inputs[:, :, 0].reshape(-1), (0, 64))
    ys = jnp.pad(inputs[:, :, 1].reshape(-1), (0, 64))
    zs = jnp.pad(inputs[:, :, 2].reshape(-1), (0, 64))

    mesh, body = _build_sc_kernel(NC, wpb, N, per_w, per_w_last // 16,
                                  H * W, CT)

    run = pl.kernel(
        body,
        mesh=mesh,
        compiler_params=pltpu.CompilerParams(needs_layout_passes=False),
        out_type=jax.ShapeDtypeStruct((B * N * CT,), jnp.float32),
        scratch_types=[
            pltpu.VMEM((per_w,), jnp.float32),
            pltpu.VMEM((per_w,), jnp.float32),
            pltpu.VMEM((per_w,), jnp.float32),
            pltpu.VMEM((B * 16,), jnp.float32),
            pltpu.VMEM((2, 64), jnp.int32),
            pltpu.VMEM((64, CT // 2), jnp.float32),
            pltpu.VMEM((64, CT // 2), jnp.float32),
            pltpu.VMEM((16 * CT,), jnp.float32),
            pltpu.SemaphoreType.DMA,
            pltpu.SemaphoreType.DMA,
        ],
    )
    sampled = run(xs, ys, zs, coef, feat0, feat1)
    return jnp.concatenate([inputs, sampled.reshape(B, N, CT)], axis=2)


# speedup vs baseline: 2.1489x; 2.1489x over previous
import jax
import jax.numpy as jnp
from jax import lax
from jax.experimental import pallas as pl
from jax.experimental.pallas import tpu as pltpu
from jax.experimental.pallas import tpu_sc as plsc


def _bcast_take(vec, idx):
    return lax.gather(
        vec, idx[:, None],
        lax.GatherDimensionNumbers(offset_dims=(), collapsed_slice_dims=(0,),
                                   start_index_map=(0,)),
        (1,), mode=lax.GatherScatterMode.PROMISE_IN_BOUNDS)


def _build_sc_kernel(n_workers, nc, per_w, n_iters, rows_per_batch, tot_cols):
    CH = 16
    HWC = tot_cols // 2
    SL = HWC // 4
    NG = tot_cols // 32
    mesh = plsc.VectorSubcoreMesh(core_axis_name="c", subcore_axis_name="s")

    def body(xs_hbm, ys_hbm, zs_hbm, coef_hbm, feat_hbm, out_hbm,
             xs_v, ys_v, zs_v, coef_v, idx_v, rows_v, out_v, sem):
        wid = lax.axis_index("s") * nc + lax.axis_index("c")
        base = wid * per_w
        b = wid // (n_workers // (coef_hbm.shape[0] // 16))
        pltpu.sync_copy(xs_hbm.at[pl.ds(base, per_w)], xs_v)
        pltpu.sync_copy(ys_hbm.at[pl.ds(base, per_w)], ys_v)
        pltpu.sync_copy(zs_hbm.at[pl.ds(base, per_w)], zs_v)
        pltpu.sync_copy(coef_hbm, coef_v)
        cvec = coef_v[pl.ds(b * 16, 16)]

        def bc(i):
            return _bcast_take(cvec, jnp.full((16,), i, jnp.int32))

        av = bc(0)
        bv = bc(1)
        cv = bc(2)
        dv = bc(3)
        ev = bc(4)
        rowoff = b * rows_per_batch
        himask = jnp.full((16,), -65536, jnp.int32)

        def iter_body(it, carry):
            o = it * CH
            x = xs_v[pl.ds(o, 16)]
            y = ys_v[pl.ds(o, 16)]
            z = zs_v[pl.ds(o, 16)] - 0.8
            wn = (av * x + bv * y) / z + cv
            hn = (dv * y) / z + ev
            wn = jnp.minimum(jnp.maximum(wn, -1.0), 1.0)
            hn = jnp.minimum(jnp.maximum(hn, -1.0), 1.0)
            ixf = ((wn + 1.0) * 56.0 - 1.0) * 0.5
            iyf = ((hn + 1.0) * 56.0 - 1.0) * 0.5

            def fl(v):
                t = v.astype(jnp.int32)
                tf = t.astype(jnp.float32)
                return t - (tf > v).astype(jnp.int32)

            ix0 = fl(ixf)
            iy0 = fl(iyf)
            wx1 = ixf - ix0.astype(jnp.float32)
            wx0 = 1.0 - wx1
            wy1 = iyf - iy0.astype(jnp.float32)
            wy0 = 1.0 - wy1
            ix1 = ix0 + 1
            iy1 = iy0 + 1
            wxa = wx0 * ((ix0 >= 0) & (ix0 <= 55)).astype(jnp.float32)
            wxb = wx1 * ((ix1 >= 0) & (ix1 <= 55)).astype(jnp.float32)
            wya = wy0 * ((iy0 >= 0) & (iy0 <= 55)).astype(jnp.float32)
            wyb = wy1 * ((iy1 >= 0) & (iy1 <= 55)).astype(jnp.float32)
            sx0 = (ix0 > 54).astype(jnp.float32)
            sx1 = (ix0 >= 0).astype(jnp.float32)
            sy0 = (iy0 > 54).astype(jnp.float32)
            sy1 = (iy0 >= 0).astype(jnp.float32)
            wx_lo = wxa * (1.0 - sx0) + wxb * (1.0 - sx1)
            wx_hi = wxa * sx0 + wxb * sx1
            wy_lo = wya * (1.0 - sy0) + wyb * (1.0 - sy1)
            wy_hi = wya * sy0 + wyb * sy1
            q00 = wy_lo * wx_lo
            q01 = wy_lo * wx_hi
            q10 = wy_hi * wx_lo
            q11 = wy_hi * wx_hi
            zero = jnp.zeros((16,), jnp.int32)
            last_p = jnp.full((16,), 54, jnp.int32)
            px = jnp.minimum(jnp.maximum(ix0, zero), last_p)
            py = jnp.minimum(jnp.maximum(iy0, zero), last_p)
            idx_v[pl.ds(0, 16)] = rowoff + py * 56 + px
            pltpu.async_copy(feat_hbm.at[idx_v], rows_v, sem).wait()

            def p_body(p, c2):
                lane = jnp.full((16,), 0, jnp.int32) + p
                wb0 = _bcast_take(q00, lane)
                wb1 = _bcast_take(q01, lane)
                wb2 = _bcast_take(q10, lane)
                wb3 = _bcast_take(q11, lane)

                @plsc.parallel_loop(0, NG, unroll=4)
                def s_body(g):
                    off = g * 16
                    r0 = rows_v[p, pl.ds(off, 16)]
                    r1 = rows_v[p, pl.ds(SL + off, 16)]
                    r2 = rows_v[p, pl.ds(2 * SL + off, 16)]
                    r3 = rows_v[p, pl.ds(3 * SL + off, 16)]

                    def lo(r):
                        return plsc.bitcast(lax.shift_left(r, 16), jnp.float32)

                    def hi(r):
                        return plsc.bitcast(r & himask, jnp.float32)

                    acc_e = (wb0 * lo(r0) + wb1 * lo(r1)
                             + wb2 * lo(r2) + wb3 * lo(r3))
                    acc_o = (wb0 * hi(r0) + wb1 * hi(r1)
                             + wb2 * hi(r2) + wb3 * hi(r3))
                    out_v[p, pl.ds(g * 32, 16)] = acc_e
                    out_v[p, pl.ds(g * 32 + 16, 16)] = acc_o

                return c2

            lax.fori_loop(0, CH, p_body, 0)
            pltpu.sync_copy(out_v, out_hbm.at[pl.ds(base + o, CH)])
            return carry

        lax.fori_loop(0, n_iters, iter_body, 0)

    return mesh, body


def kernel(resolution, img_features, inputs, camK):
    B, N, _ = inputs.shape
    L, _, C, H, W = img_features.shape
    CT = L * C

    info = plsc.get_sparse_core_info()
    NC, NS = info.num_cores, info.num_subcores
    NW = NC * NS
    wpb = NW // B
    per_w = -(-N // (wpb * 16)) * 16
    n_iters = per_w // 16
    Npad = per_w * wpb

    scale = 256.0 / 1920.0
    k = camK * scale
    hr = (resolution - 1.0) / 2.0
    hr0, hr1 = hr[0], hr[1]
    a = -k[:, 0, 0] / hr0
    bb = -k[:, 0, 1] / hr0
    c = (k[:, 0, 2] - hr0) / hr0
    d = k[:, 1, 1] / hr1
    e = (k[:, 1, 2] - hr1) / hr1
    z3 = jnp.zeros_like(a)
    coef = jnp.stack([a, bb, c, d, e] + [z3] * 11, axis=1).reshape(-1)

    F = jnp.transpose(img_features, (1, 3, 4, 0, 2)).reshape(B, H, W, CT)
    Fx = jnp.concatenate([F[:, :, 1:], F[:, :, W - 1:]], axis=2)
    Fy = jnp.concatenate([F[:, 1:], F[:, H - 1:]], axis=1)
    Fxy = jnp.concatenate([Fy[:, :, 1:], Fy[:, :, W - 1:]], axis=2)
    Q = jnp.concatenate([F, Fx, Fy, Fxy], axis=-1).astype(jnp.bfloat16)
    Q = Q.reshape(B, H, W, 4, CT // 32, 2, 16)
    Q = jnp.swapaxes(Q, -1, -2)
    Qi = lax.bitcast_convert_type(
        Q.reshape(B * H * W, 2 * CT, 2), jnp.int32)

    inp_p = jnp.pad(inputs, ((0, 0), (0, Npad - N), (0, 0)))
    xs = inp_p[:, :, 0].reshape(-1)
    ys = inp_p[:, :, 1].reshape(-1)
    zs = inp_p[:, :, 2].reshape(-1)

    mesh, body = _build_sc_kernel(NW, NC, per_w, n_iters, H * W, CT)

    run = pl.kernel(
        body,
        mesh=mesh,
        compiler_params=pltpu.CompilerParams(needs_layout_passes=False),
        out_type=jax.ShapeDtypeStruct((B * Npad, CT), jnp.float32),
        scratch_types=[
            pltpu.VMEM((per_w,), jnp.float32),
            pltpu.VMEM((per_w,), jnp.float32),
            pltpu.VMEM((per_w,), jnp.float32),
            pltpu.VMEM((B * 16,), jnp.float32),
            pltpu.VMEM((16,), jnp.int32),
            pltpu.VMEM((16, 2 * CT), jnp.int32),
            pltpu.VMEM((16, CT), jnp.float32),
            pltpu.SemaphoreType.DMA,
        ],
    )
    sampled = run(xs, ys, zs, coef, Qi)
    sampled = sampled.reshape(B, Npad, CT)[:, :N, :]
    return jnp.concatenate([inputs, sampled], axis=2)
